# baseline (speedup 1.0000x reference)
import jax
import jax.numpy as jnp
from jax.experimental import pallas as pl
from jax.experimental.pallas import tpu as pltpu
from jax.experimental.pallas import tpu_sc as plsc

_D = 32

_mesh = plsc.VectorSubcoreMesh(core_axis_name="core", subcore_axis_name="subcore")


@jax.jit
def _gather(table, src):
  n_rows, row_len = src.shape

  @pl.kernel(
      out_type=jax.ShapeDtypeStruct((n_rows, row_len, _D), jnp.float32),
      mesh=_mesh,
      compiler_params=pltpu.CompilerParams(use_tc_tiling_on_sc=False),
  )
  def k(table_hbm, i_hbm, o_hbm):
    def body(i_vmem, o_vmem):
      pltpu.sync_copy(table_hbm.at[i_vmem.at[0]], o_vmem.at[0])

    pltpu.emit_pipeline(
        body,
        grid=(n_rows,),
        in_specs=[pl.BlockSpec((1, row_len), index_map=lambda i: (i, 0))],
        out_specs=[pl.BlockSpec((1, row_len, _D), index_map=lambda i: (i, 0, 0))],
        core_axis_name=("core", "subcore"),
        dimension_semantics=(pltpu.PARALLEL,),
    )(i_hbm, o_hbm)

  return k(table, src)


def kernel(src, table):
  return _gather(table, src)


# speedup vs baseline: 1.2636x; 1.2636x over previous
import jax
import jax.numpy as jnp
from jax.experimental import pallas as pl
from jax.experimental.pallas import tpu as pltpu
from jax.experimental.pallas import tpu_sc as plsc

_D = 32
_C = 128

_mesh = plsc.VectorSubcoreMesh(core_axis_name="core", subcore_axis_name="subcore")


@jax.jit
def _gather(table, src_t):
  n_tok, n_batch = src_t.shape

  @pl.kernel(
      out_type=jax.ShapeDtypeStruct((n_tok, n_batch, _D), jnp.float32),
      mesh=_mesh,
      compiler_params=pltpu.CompilerParams(use_tc_tiling_on_sc=False),
  )
  def k(table_hbm, i_hbm, o_hbm):
    def body(i_vmem, o_vmem):
      pltpu.sync_copy(table_hbm.at[i_vmem.at[0]], o_vmem.at[0])

    pltpu.emit_pipeline(
        body,
        grid=(n_tok, n_batch // _C),
        in_specs=[pl.BlockSpec((1, _C), index_map=lambda t, c: (t, c))],
        out_specs=[pl.BlockSpec((1, _C, _D), index_map=lambda t, c: (t, c, 0))],
        core_axis_name=("core", "subcore"),
        dimension_semantics=(pltpu.PARALLEL, pltpu.PARALLEL),
    )(i_hbm, o_hbm)

  return k(table, src_t)


def kernel(src, table):
  out_t = _gather(table, src.T)
  return out_t.transpose(1, 0, 2)
